# baseline (speedup 1.0000x reference)
import functools

import jax
import jax.numpy as jnp
from jax import lax
from jax.experimental import pallas as pl
from jax.experimental.pallas import tpu as pltpu
from jax.experimental.pallas import tpu_sc as plsc

BATCH = 16384
EMBED = 128
NC = 2
NS = 16
L = 16
NW = NC * NS
B_PER_W = BATCH // NW
CHUNK = 128
N_CHUNKS = B_PER_W // CHUNK
GROUPS = CHUNK // L


def _body(u_ids_hbm, i_ids_hbm, usr_hbm, itm_hbm, ent_hbm, pad_hbm, out_hbm,
          uidx_v, iidx_v, eidx_v, urows_v, irows_v, erows_v, score_v, sem):
    wid = lax.axis_index("s") * NC + lax.axis_index("c")
    base = wid * B_PER_W

    for c in range(N_CHUNKS):
        off = base + c * CHUNK
        pltpu.sync_copy(u_ids_hbm.at[pl.ds(off, CHUNK)], uidx_v)
        pltpu.sync_copy(i_ids_hbm.at[pl.ds(off, CHUNK)], iidx_v)
        pltpu.async_copy(pad_hbm.at[iidx_v], eidx_v, sem).wait()
        pltpu.async_copy(usr_hbm.at[uidx_v], urows_v, sem).wait()
        pltpu.async_copy(itm_hbm.at[iidx_v], irows_v, sem).wait()
        pltpu.async_copy(ent_hbm.at[eidx_v], erows_v, sem).wait()

        for g in range(GROUPS):
            row_idx = jnp.full((L,), g * L, jnp.int32) + lax.iota(jnp.int32, L)

            def col(k, acc):
                col_idx = jnp.full((L,), k, jnp.int32)
                u = plsc.load_gather(urows_v, [row_idx, col_idx])
                iv = plsc.load_gather(irows_v, [row_idx, col_idx])
                e = plsc.load_gather(erows_v, [row_idx, col_idx])
                return acc + u * (iv + e)

            acc = lax.fori_loop(0, EMBED, col, jnp.zeros((L,), jnp.float32))
            score_v[pl.ds(g * L, L)] = acc

        pltpu.sync_copy(score_v, out_hbm.at[pl.ds(off, CHUNK)])


@jax.jit
def _run(u_ids, i_ids, usr_emb, itm_emb, ent_emb, padding_items):
    mesh = plsc.VectorSubcoreMesh(core_axis_name="c", subcore_axis_name="s")
    return pl.kernel(
        _body,
        mesh=mesh,
        compiler_params=pltpu.CompilerParams(needs_layout_passes=False),
        out_type=jax.ShapeDtypeStruct((BATCH,), jnp.float32),
        scratch_types=[
            pltpu.VMEM((CHUNK,), jnp.int32),
            pltpu.VMEM((CHUNK,), jnp.int32),
            pltpu.VMEM((CHUNK,), jnp.int32),
            pltpu.VMEM((CHUNK, EMBED), jnp.float32),
            pltpu.VMEM((CHUNK, EMBED), jnp.float32),
            pltpu.VMEM((CHUNK, EMBED), jnp.float32),
            pltpu.VMEM((CHUNK,), jnp.float32),
            pltpu.SemaphoreType.DMA,
        ],
    )(u_ids, i_ids, usr_emb, itm_emb, ent_emb, padding_items)


def kernel(u_ids, i_ids, usr_emb, itm_emb, ent_emb, padding_items):
    u_ids = jnp.asarray(u_ids, jnp.int32).reshape(BATCH)
    i_ids = jnp.asarray(i_ids, jnp.int32).reshape(BATCH)
    return _run(u_ids, i_ids, usr_emb, itm_emb, ent_emb, padding_items)


# speedup vs baseline: 1.1338x; 1.1338x over previous
import functools

import jax
import jax.numpy as jnp
from jax import lax
from jax.experimental import pallas as pl
from jax.experimental.pallas import tpu as pltpu
from jax.experimental.pallas import tpu_sc as plsc

BATCH = 16384
EMBED = 128
NC = 2
NS = 16
L = 16
NW = NC * NS
B_PER_W = BATCH // NW
CHUNK = 64
N_CHUNKS = B_PER_W // CHUNK
GROUPS = CHUNK // L
NSLOT = 4


def _body(u_ids_hbm, i_ids_hbm, usr_hbm, itm_hbm, ent_hbm, pad_hbm, out_hbm,
          *scratch):
    uidx = scratch[0:N_CHUNKS]
    iidx = scratch[N_CHUNKS:2 * N_CHUNKS]
    eidx = scratch[2 * N_CHUNKS:3 * N_CHUNKS]
    urows, irows, erows, score_v = scratch[3 * N_CHUNKS:3 * N_CHUNKS + 4]
    sem_ids = scratch[3 * N_CHUNKS + 4]
    sem_out = scratch[3 * N_CHUNKS + 5]
    slot_sems = scratch[3 * N_CHUNKS + 6:]

    wid = lax.axis_index("s") * NC + lax.axis_index("c")
    base = wid * B_PER_W

    id_cps = []
    for c in range(N_CHUNKS):
        off = base + c * CHUNK
        id_cps.append(pltpu.async_copy(u_ids_hbm.at[pl.ds(off, CHUNK)], uidx[c], sem_ids))
        id_cps.append(pltpu.async_copy(i_ids_hbm.at[pl.ds(off, CHUNK)], iidx[c], sem_ids))
    for cp in id_cps:
        cp.wait()
    e_cps = [pltpu.async_copy(pad_hbm.at[iidx[c]], eidx[c], sem_ids)
             for c in range(N_CHUNKS)]
    for cp in e_cps:
        cp.wait()

    def fire(c):
        s = c % NSLOT
        sem = slot_sems[s]
        return (pltpu.async_copy(usr_hbm.at[uidx[c]], urows.at[s], sem),
                pltpu.async_copy(itm_hbm.at[iidx[c]], irows.at[s], sem),
                pltpu.async_copy(ent_hbm.at[eidx[c]], erows.at[s], sem))

    inflight = [fire(c) for c in range(NSLOT - 1)]
    out_cps = []
    for c in range(N_CHUNKS):
        for cp in inflight[0]:
            cp.wait()
        inflight = inflight[1:]
        s = c % NSLOT

        for g in range(GROUPS):
            row_idx = jnp.full((L,), g * L, jnp.int32) + lax.iota(jnp.int32, L)

            def col(k, acc):
                col_idx = jnp.full((L,), k, jnp.int32)
                u = plsc.load_gather(urows.at[s], [row_idx, col_idx])
                iv = plsc.load_gather(irows.at[s], [row_idx, col_idx])
                e = plsc.load_gather(erows.at[s], [row_idx, col_idx])
                return acc + u * (iv + e)

            acc = lax.fori_loop(0, EMBED, col, jnp.zeros((L,), jnp.float32))
            score_v[c, pl.ds(g * L, L)] = acc

        if c + NSLOT - 1 < N_CHUNKS:
            inflight.append(fire(c + NSLOT - 1))
        out_cps.append(pltpu.async_copy(
            score_v.at[c], out_hbm.at[pl.ds(base + c * CHUNK, CHUNK)], sem_out))
    for cp in out_cps:
        cp.wait()


@jax.jit
def _run(u_ids, i_ids, usr_emb, itm_emb, ent_emb, padding_items):
    mesh = plsc.VectorSubcoreMesh(core_axis_name="c", subcore_axis_name="s")
    idx_scratch = [pltpu.VMEM((CHUNK,), jnp.int32) for _ in range(3 * N_CHUNKS)]
    return pl.kernel(
        _body,
        mesh=mesh,
        compiler_params=pltpu.CompilerParams(needs_layout_passes=False),
        out_type=jax.ShapeDtypeStruct((BATCH,), jnp.float32),
        scratch_types=idx_scratch + [
            pltpu.VMEM((NSLOT, CHUNK, EMBED), jnp.float32),
            pltpu.VMEM((NSLOT, CHUNK, EMBED), jnp.float32),
            pltpu.VMEM((NSLOT, CHUNK, EMBED), jnp.float32),
            pltpu.VMEM((N_CHUNKS, CHUNK), jnp.float32),
            pltpu.SemaphoreType.DMA,
            pltpu.SemaphoreType.DMA,
        ] + [pltpu.SemaphoreType.DMA for _ in range(NSLOT)],
    )(u_ids, i_ids, usr_emb, itm_emb, ent_emb, padding_items)


def kernel(u_ids, i_ids, usr_emb, itm_emb, ent_emb, padding_items):
    u_ids = jnp.asarray(u_ids, jnp.int32).reshape(BATCH)
    i_ids = jnp.asarray(i_ids, jnp.int32).reshape(BATCH)
    return _run(u_ids, i_ids, usr_emb, itm_emb, ent_emb, padding_items)


# speedup vs baseline: 3.7763x; 3.3308x over previous
import functools

import jax
import jax.numpy as jnp
from jax import lax
from jax.experimental import pallas as pl
from jax.experimental.pallas import tpu as pltpu
from jax.experimental.pallas import tpu_sc as plsc

BATCH = 16384
EMBED = 128
NC = 2
NS = 16
L = 16
NW = NC * NS
B_PER_W = BATCH // NW
CHUNK = 128
N_CHUNKS = B_PER_W // CHUNK
GROUPS = CHUNK // L
NSLOT = 2
SEGS = EMBED // L


def _body(u_ids_hbm, i_ids_hbm, usr_hbm, itm_hbm, ent_hbm, pad_hbm, out_hbm,
          *scratch):
    uidx = scratch[0:N_CHUNKS]
    iidx = scratch[N_CHUNKS:2 * N_CHUNKS]
    eidx = scratch[2 * N_CHUNKS:3 * N_CHUNKS]
    urows, irows, erows, score_v = scratch[3 * N_CHUNKS:3 * N_CHUNKS + 4]
    sem_ids = scratch[3 * N_CHUNKS + 4]
    sem_out = scratch[3 * N_CHUNKS + 5]
    slot_sems = scratch[3 * N_CHUNKS + 6:]

    wid = lax.axis_index("s") * NC + lax.axis_index("c")
    base = wid * B_PER_W

    lane = lax.iota(jnp.int32, L)
    masks = [lane == j for j in range(L)]

    id_cps = []
    for c in range(N_CHUNKS):
        off = base + c * CHUNK
        id_cps.append(pltpu.async_copy(u_ids_hbm.at[pl.ds(off, CHUNK)], uidx[c], sem_ids))
        id_cps.append(pltpu.async_copy(i_ids_hbm.at[pl.ds(off, CHUNK)], iidx[c], sem_ids))
    for cp in id_cps:
        cp.wait()
    e_cps = [pltpu.async_copy(pad_hbm.at[iidx[c]], eidx[c], sem_ids)
             for c in range(N_CHUNKS)]
    for cp in e_cps:
        cp.wait()

    def fire(c):
        s = c % NSLOT
        sem = slot_sems[s]
        return (pltpu.async_copy(usr_hbm.at[uidx[c]], urows.at[s], sem),
                pltpu.async_copy(itm_hbm.at[iidx[c]], irows.at[s], sem),
                pltpu.async_copy(ent_hbm.at[eidx[c]], erows.at[s], sem))

    inflight = [fire(c) for c in range(NSLOT)]
    out_cps = []
    for c in range(N_CHUNKS):
        for cp in inflight[0]:
            cp.wait()
        inflight = inflight[1:]
        s = c % NSLOT

        def row_body(r, score_vec):
            acc0 = jnp.zeros((L,), jnp.float32)
            acc1 = jnp.zeros((L,), jnp.float32)
            for seg in range(SEGS):
                u = urows[s, r, pl.ds(seg * L, L)]
                iv = irows[s, r, pl.ds(seg * L, L)]
                e = erows[s, r, pl.ds(seg * L, L)]
                if seg % 2 == 0:
                    acc0 = acc0 + u * (iv + e)
                else:
                    acc1 = acc1 + u * (iv + e)
            sc = jnp.sum(acc0 + acc1)
            j = jnp.bitwise_and(r, L - 1)
            score_vec = jnp.where(lane == j, sc, score_vec)

            @pl.when(j == L - 1)
            def _store():
                score_v[c, pl.ds((r // L) * L, L)] = score_vec

            return score_vec

        lax.fori_loop(0, CHUNK, row_body, jnp.zeros((L,), jnp.float32),
                      unroll=2)

        if c + NSLOT < N_CHUNKS:
            inflight.append(fire(c + NSLOT))
        out_cps.append(pltpu.async_copy(
            score_v.at[c], out_hbm.at[pl.ds(base + c * CHUNK, CHUNK)], sem_out))
    for cp in out_cps:
        cp.wait()


@jax.jit
def _run(u_ids, i_ids, usr_emb, itm_emb, ent_emb, padding_items):
    mesh = plsc.VectorSubcoreMesh(core_axis_name="c", subcore_axis_name="s")
    idx_scratch = [pltpu.VMEM((CHUNK,), jnp.int32) for _ in range(3 * N_CHUNKS)]
    return pl.kernel(
        _body,
        mesh=mesh,
        compiler_params=pltpu.CompilerParams(needs_layout_passes=False),
        out_type=jax.ShapeDtypeStruct((BATCH,), jnp.float32),
        scratch_types=idx_scratch + [
            pltpu.VMEM((NSLOT, CHUNK, EMBED), jnp.float32),
            pltpu.VMEM((NSLOT, CHUNK, EMBED), jnp.float32),
            pltpu.VMEM((NSLOT, CHUNK, EMBED), jnp.float32),
            pltpu.VMEM((N_CHUNKS, CHUNK), jnp.float32),
            pltpu.SemaphoreType.DMA,
            pltpu.SemaphoreType.DMA,
        ] + [pltpu.SemaphoreType.DMA for _ in range(NSLOT)],
    )(u_ids, i_ids, usr_emb, itm_emb, ent_emb, padding_items)


def kernel(u_ids, i_ids, usr_emb, itm_emb, ent_emb, padding_items):
    u_ids = jnp.asarray(u_ids, jnp.int32).reshape(BATCH)
    i_ids = jnp.asarray(i_ids, jnp.int32).reshape(BATCH)
    return _run(u_ids, i_ids, usr_emb, itm_emb, ent_emb, padding_items)
